# baseline (speedup 1.0000x reference)
import functools

import jax
import jax.numpy as jnp
from jax import lax
from jax.experimental import pallas as pl
from jax.experimental.pallas import tpu as pltpu

R = 4096
WN = 64
W = 1032
PAD = 2048


def _fused_kernel(carr, ids_ref, prev_ref, x_ref, emb_ref, wsrc_ref, bsrc_ref,
                  wtgt_ref, btgt_ref, xout_ref, iout_ref, p_sc, sums_sc,
                  cnts_sc, *, nb, H, S):
    i = pl.program_id(0)
    c = carr[i]
    base = (c // 8) * 8
    off = (c - base).astype(jnp.float32)
    nbound = carr[i + 1] - c

    @pl.when(i == 0)
    def _init():
        p_sc[...] = jnp.zeros_like(p_sc)
        sums_sc[...] = jnp.zeros_like(sums_sc)
        cnts_sc[...] = jnp.zeros_like(cnts_sc)
        p_sc[0:S, :] = jnp.dot(emb_ref[...].astype(jnp.bfloat16),
                               wsrc_ref[H:2 * H, :].astype(jnp.bfloat16),
                               preferred_element_type=jnp.float32)

    bnd = (ids_ref[0] != prev_ref[0]).astype(jnp.float32)
    lane = lax.broadcasted_iota(jnp.int32, (1, R), 1)
    seg_row = bnd
    shift = 1
    while shift < R:
        rolled = pltpu.roll(seg_row, shift, 1)
        seg_row = seg_row + jnp.where(lane >= shift, rolled, 0.0)
        shift *= 2
    rel_r = (seg_row + off).astype(jnp.int32)

    xb = x_ref[...].astype(jnp.bfloat16)
    ones_rc = jnp.ones((R, 128), dtype=jnp.bfloat16)
    main_part = (jnp.dot(xb, wsrc_ref[0:H, :].astype(jnp.bfloat16),
                         preferred_element_type=jnp.float32)
                 + bsrc_ref[...])

    def _window_body(width):
        def go():
            onehot_t = (lax.broadcasted_iota(jnp.int32, (width, R), 0)
                        == rel_r).astype(jnp.bfloat16)
            p_win = p_sc[pl.ds(base, width), :].astype(jnp.bfloat16)
            gathered = lax.dot_general(
                onehot_t, p_win, (((0,), (0,)), ((), ())),
                preferred_element_type=jnp.float32)
            xout_ref[...] = main_part + gathered
            sums_sc[pl.ds(base, width), :] += jnp.dot(
                onehot_t, xb, preferred_element_type=jnp.float32)
            cnts_sc[pl.ds(base, width), :] += jnp.dot(
                onehot_t, ones_rc, preferred_element_type=jnp.float32)
        return go

    pl.when(nbound <= WN - 8)(_window_body(WN))
    pl.when(nbound > WN - 8)(_window_body(W))

    @pl.when(i == nb - 1)
    def _finish():
        n_runs = carr[nb] + 1
        means = sums_sc[0:S, :] / jnp.maximum(cnts_sc[0:S, 0:1], 1.0)
        sidx = lax.broadcasted_iota(jnp.int32, (S, 1), 0)
        emb = emb_ref[...]
        second = jnp.where(sidx < (n_runs - 1), means, emb)
        iout_ref[...] = (
            jnp.dot(emb.astype(jnp.bfloat16),
                    wtgt_ref[0:H, :].astype(jnp.bfloat16),
                    preferred_element_type=jnp.float32)
            + jnp.dot(second.astype(jnp.bfloat16),
                      wtgt_ref[H:2 * H, :].astype(jnp.bfloat16),
                      preferred_element_type=jnp.float32)
            + btgt_ref[...])


@jax.jit
def kernel(x_src, imputed_embs, src_ids, W_src, b_src, W_tgt, b_tgt):
    N, H = x_src.shape
    S = imputed_embs.shape[0]
    nb = N // R

    prev_ids = jnp.concatenate([src_ids[:1], src_ids[:-1]])
    bnd2 = (src_ids.reshape(nb, R) != prev_ids.reshape(nb, R))
    blk_counts = jnp.sum(bnd2.astype(jnp.int32), axis=1)
    carr = jnp.concatenate(
        [jnp.zeros((1,), jnp.int32), jnp.cumsum(blk_counts)]).astype(jnp.int32)

    ids3 = src_ids.reshape(nb, 1, R)
    prev3 = prev_ids.reshape(nb, 1, R)

    grid_spec = pltpu.PrefetchScalarGridSpec(
        num_scalar_prefetch=1,
        grid=(nb,),
        in_specs=[
            pl.BlockSpec((1, 1, R), lambda i, c: (i, 0, 0)),
            pl.BlockSpec((1, 1, R), lambda i, c: (i, 0, 0)),
            pl.BlockSpec((R, H), lambda i, c: (i, 0)),
            pl.BlockSpec((S, H), lambda i, c: (0, 0)),
            pl.BlockSpec((2 * H, H), lambda i, c: (0, 0)),
            pl.BlockSpec((1, H), lambda i, c: (0, 0)),
            pl.BlockSpec((2 * H, H), lambda i, c: (0, 0)),
            pl.BlockSpec((1, H), lambda i, c: (0, 0)),
        ],
        out_specs=[
            pl.BlockSpec((R, H), lambda i, c: (i, 0)),
            pl.BlockSpec((S, H), lambda i, c: (0, 0)),
        ],
        scratch_shapes=[
            pltpu.VMEM((PAD, H), jnp.float32),
            pltpu.VMEM((PAD, H), jnp.float32),
            pltpu.VMEM((PAD, 128), jnp.float32),
        ],
    )

    x_out, imputed_out = pl.pallas_call(
        functools.partial(_fused_kernel, nb=nb, H=H, S=S),
        grid_spec=grid_spec,
        out_shape=[
            jax.ShapeDtypeStruct((N, H), jnp.float32),
            jax.ShapeDtypeStruct((S, H), jnp.float32),
        ],
        compiler_params=pltpu.CompilerParams(
            dimension_semantics=("arbitrary",)),
    )(carr, ids3, prev3, x_src, imputed_embs, W_src,
      b_src.reshape(1, H), W_tgt, b_tgt.reshape(1, H))
    return (x_out, imputed_out)


# speedup vs baseline: 1.9379x; 1.9379x over previous
import functools

import jax
import jax.numpy as jnp
from jax import lax
from jax.experimental import pallas as pl
from jax.experimental.pallas import tpu as pltpu

R = 2048
WN = 64
W = 1032
PAD = 2048


def _fused_kernel(carr, ids_ref, prev_ref, x_ref, emb_ref, wsrc_ref, bsrc_ref,
                  wtgt_ref, btgt_ref, xout_ref, iout_ref, p_sc, sums_sc,
                  cnts_sc, *, nb, H, S):
    i = pl.program_id(0)
    c = carr[i]
    base = (c // 8) * 8
    off = (c - base).astype(jnp.float32)
    nbound = carr[i + 1] - c

    @pl.when(i == 0)
    def _init():
        p_sc[...] = jnp.zeros_like(p_sc)
        sums_sc[...] = jnp.zeros_like(sums_sc)
        cnts_sc[...] = jnp.zeros_like(cnts_sc)
        p_sc[0:S, :] = jnp.dot(emb_ref[...].astype(jnp.bfloat16),
                               wsrc_ref[H:2 * H, :].astype(jnp.bfloat16),
                               preferred_element_type=jnp.float32)

    bnd = (ids_ref[0] != prev_ref[0]).astype(jnp.float32)
    lane = lax.broadcasted_iota(jnp.int32, (1, R), 1)
    seg_row = bnd
    shift = 1
    while shift < R:
        rolled = pltpu.roll(seg_row, shift, 1)
        seg_row = seg_row + jnp.where(lane >= shift, rolled, 0.0)
        shift *= 2
    rel_r = (seg_row + off).astype(jnp.int32)

    xb = x_ref[...].astype(jnp.bfloat16)
    ones_rc = jnp.ones((R, 128), dtype=jnp.bfloat16)
    main_part = (jnp.dot(xb, wsrc_ref[0:H, :].astype(jnp.bfloat16),
                         preferred_element_type=jnp.float32)
                 + bsrc_ref[...])

    def _window_body(width):
        def go():
            onehot_t = (lax.broadcasted_iota(jnp.int32, (width, R), 0)
                        == rel_r).astype(jnp.bfloat16)
            p_win = p_sc[pl.ds(base, width), :].astype(jnp.bfloat16)
            gathered = lax.dot_general(
                onehot_t, p_win, (((0,), (0,)), ((), ())),
                preferred_element_type=jnp.float32)
            xout_ref[...] = main_part + gathered
            sums_sc[pl.ds(base, width), :] += jnp.dot(
                onehot_t, xb, preferred_element_type=jnp.float32)
            cnts_sc[pl.ds(base, width), :] += jnp.dot(
                onehot_t, ones_rc, preferred_element_type=jnp.float32)
        return go

    pl.when(nbound <= WN - 8)(_window_body(WN))
    pl.when(nbound > WN - 8)(_window_body(W))

    @pl.when(i == nb - 1)
    def _finish():
        n_runs = carr[nb] + 1
        means = sums_sc[0:S, :] / jnp.maximum(cnts_sc[0:S, 0:1], 1.0)
        sidx = lax.broadcasted_iota(jnp.int32, (S, 1), 0)
        emb = emb_ref[...]
        second = jnp.where(sidx < (n_runs - 1), means, emb)
        iout_ref[...] = (
            jnp.dot(emb.astype(jnp.bfloat16),
                    wtgt_ref[0:H, :].astype(jnp.bfloat16),
                    preferred_element_type=jnp.float32)
            + jnp.dot(second.astype(jnp.bfloat16),
                      wtgt_ref[H:2 * H, :].astype(jnp.bfloat16),
                      preferred_element_type=jnp.float32)
            + btgt_ref[...])


@jax.jit
def kernel(x_src, imputed_embs, src_ids, W_src, b_src, W_tgt, b_tgt):
    N, H = x_src.shape
    S = imputed_embs.shape[0]
    nb = N // R

    prev_ids = jnp.concatenate([src_ids[:1], src_ids[:-1]])
    bnd2 = (src_ids.reshape(nb, R) != prev_ids.reshape(nb, R))
    blk_counts = jnp.sum(bnd2.astype(jnp.int32), axis=1)
    carr = jnp.concatenate(
        [jnp.zeros((1,), jnp.int32), jnp.cumsum(blk_counts)]).astype(jnp.int32)

    ids3 = src_ids.reshape(nb, 1, R)
    prev3 = prev_ids.reshape(nb, 1, R)

    grid_spec = pltpu.PrefetchScalarGridSpec(
        num_scalar_prefetch=1,
        grid=(nb,),
        in_specs=[
            pl.BlockSpec((1, 1, R), lambda i, c: (i, 0, 0)),
            pl.BlockSpec((1, 1, R), lambda i, c: (i, 0, 0)),
            pl.BlockSpec((R, H), lambda i, c: (i, 0)),
            pl.BlockSpec((S, H), lambda i, c: (0, 0)),
            pl.BlockSpec((2 * H, H), lambda i, c: (0, 0)),
            pl.BlockSpec((1, H), lambda i, c: (0, 0)),
            pl.BlockSpec((2 * H, H), lambda i, c: (0, 0)),
            pl.BlockSpec((1, H), lambda i, c: (0, 0)),
        ],
        out_specs=[
            pl.BlockSpec((R, H), lambda i, c: (i, 0)),
            pl.BlockSpec((S, H), lambda i, c: (0, 0)),
        ],
        scratch_shapes=[
            pltpu.VMEM((PAD, H), jnp.float32),
            pltpu.VMEM((PAD, H), jnp.float32),
            pltpu.VMEM((PAD, 128), jnp.float32),
        ],
    )

    x_out, imputed_out = pl.pallas_call(
        functools.partial(_fused_kernel, nb=nb, H=H, S=S),
        grid_spec=grid_spec,
        out_shape=[
            jax.ShapeDtypeStruct((N, H), jnp.float32),
            jax.ShapeDtypeStruct((S, H), jnp.float32),
        ],
        compiler_params=pltpu.CompilerParams(
            dimension_semantics=("arbitrary",)),
    )(carr, ids3, prev3, x_src, imputed_embs, W_src,
      b_src.reshape(1, H), W_tgt, b_tgt.reshape(1, H))
    return (x_out, imputed_out)
